# Initial kernel scaffold; baseline (speedup 1.0000x reference)
import jax
import jax.numpy as jnp
from jax.experimental import pallas as pl


def kernel(final_v, ff, padded_tensor):
    raise NotImplementedError("write your pallas kernel here")


# speedup vs baseline: 6.3844x; 6.3844x over previous
import jax
import jax.numpy as jnp
from jax import lax
from jax.experimental import pallas as pl
from jax.experimental.pallas import tpu as pltpu
from jax.experimental.pallas import tpu_sc as plsc

_W = 1000.0
NC, NS, L = 2, 16, 16
NW = NC * NS
FACES = 256 * 512
FPW = FACES // NW
ROWS_PW = FPW // 512
CHUNKS = 512 // L


def _sc_body(fvf_hbm, fff_hbm, padded_hbm, out_hbm,
             idx_faces, b0, b1, b2, v0, v1, v2,
             c00, c01, c02, c10, c11, c12, c20, c21, c22,
             d00, d01, d02, d10, d11, d12, d20, d21, d22,
             obuf, sem):
  bs = (b0, b1, b2)
  vs = (v0, v1, v2)
  cs = ((c00, c01, c02), (c10, c11, c12), (c20, c21, c22))
  ds = ((d00, d01, d02), (d10, d11, d12), (d20, d21, d22))
  wid = lax.axis_index("s") * NC + lax.axis_index("c")
  base = wid * FPW

  pltpu.sync_copy(padded_hbm.at[pl.ds(base, FPW)], idx_faces)

  def build_b(j, carry):
    t = idx_faces[pl.ds(j * L, L)] * 3
    for k in range(3):
      bs[k][pl.ds(j * L, L)] = t + k
    return carry

  lax.fori_loop(0, FPW // L, build_b, 0, unroll=2)

  copies = [pltpu.make_async_copy(fff_hbm.at[bs[k]], vs[k], sem)
            for k in range(3)]
  for c in copies:
    c.start()
  for c in copies:
    c.wait()

  def build_c(k):
    def body(j, carry):
      t = vs[k][pl.ds(j * L, L)] * 3
      for c in range(3):
        cs[k][c][pl.ds(j * L, L)] = t + c
      return carry
    return body

  for k in range(3):
    lax.fori_loop(0, FPW // L, build_c(k), 0, unroll=2)

  copies = [pltpu.make_async_copy(fvf_hbm.at[cs[k][c]], ds[k][c], sem)
            for k in range(3) for c in range(3)]
  for c in copies:
    c.start()
  for c in copies:
    c.wait()

  def row_body(r, loss_acc):
    def chunk_body(j, acc):
      sl = pl.ds((r * CHUNKS + j) * L, L)
      x0, y0, z0 = ds[0][0][sl], ds[0][1][sl], ds[0][2][sl]
      x1, y1, z1 = ds[1][0][sl], ds[1][1][sl], ds[1][2][sl]
      x2, y2, z2 = ds[2][0][sl], ds[2][1][sl], ds[2][2][sl]
      sv = (x0 * (y1 * z2 - z1 * y2)
            + y0 * (z1 * x2 - x1 * z2)
            + z0 * (x1 * y2 - y1 * x2))
      return acc + sv

    acc = lax.fori_loop(0, CHUNKS, chunk_body,
                        jnp.zeros((L,), jnp.float32), unroll=2)
    return loss_acc + jnp.abs(jnp.sum(acc))

  loss = lax.fori_loop(0, ROWS_PW, row_body, jnp.float32(0.0))

  lane = lax.iota(jnp.int32, L)
  obuf[...] = jnp.where(lane == 0, loss * _W, jnp.float32(0.0))
  pltpu.sync_copy(obuf, out_hbm.at[wid])


@jax.jit
def kernel(final_v, ff, padded_tensor):
  fv_flat = final_v.reshape(-1)
  ff_flat = ff.reshape(-1)
  padded_flat = padded_tensor.reshape(-1)
  mesh = plsc.VectorSubcoreMesh(core_axis_name="c", subcore_axis_name="s")
  i32buf = pltpu.VMEM((FPW,), jnp.int32)
  f32buf = pltpu.VMEM((FPW,), jnp.float32)
  partials = pl.kernel(
      _sc_body,
      out_type=jax.ShapeDtypeStruct((NW, L), jnp.float32),
      mesh=mesh,
      scratch_types=(
          [i32buf] * 7 + [i32buf] * 9 + [f32buf] * 9
          + [pltpu.VMEM((L,), jnp.float32), pltpu.SemaphoreType.DMA]
      ),
      compiler_params=pltpu.CompilerParams(needs_layout_passes=False),
  )(fv_flat, ff_flat, padded_flat)
  return jnp.sum(partials)


# speedup vs baseline: 15.4536x; 2.4205x over previous
import jax
import jax.numpy as jnp
from jax import lax
from jax.experimental import pallas as pl
from jax.experimental.pallas import tpu as pltpu
from jax.experimental.pallas import tpu_sc as plsc

_W = 1000.0
NC, NS, L = 2, 16, 16
NW = NC * NS
FACES = 256 * 512
FPW = FACES // NW
ROWS_PW = FPW // 512
CHUNKS = 512 // L


def _sc_body(vx, vy, vz, f0, f1, f2, padded_hbm, out_hbm,
             idx_faces, v0, v1, v2,
             d00, d01, d02, d10, d11, d12, d20, d21, d22,
             obuf, sem):
  vtabs = (vx, vy, vz)
  ftabs = (f0, f1, f2)
  vids = (v0, v1, v2)
  ds = ((d00, d01, d02), (d10, d11, d12), (d20, d21, d22))
  wid = lax.axis_index("s") * NC + lax.axis_index("c")
  base = wid * FPW

  pltpu.sync_copy(padded_hbm.at[pl.ds(base, FPW)], idx_faces)

  copies = [pltpu.make_async_copy(ftabs[k].at[idx_faces], vids[k], sem)
            for k in range(3)]
  for c in copies:
    c.start()
  for c in copies:
    c.wait()

  copies = [pltpu.make_async_copy(vtabs[c].at[vids[k]], ds[k][c], sem)
            for k in range(3) for c in range(3)]
  for c in copies:
    c.start()
  for c in copies:
    c.wait()

  def row_body(r, loss_acc):
    def chunk_body(j, acc):
      sl = pl.ds((r * CHUNKS + j) * L, L)
      x0, y0, z0 = ds[0][0][sl], ds[0][1][sl], ds[0][2][sl]
      x1, y1, z1 = ds[1][0][sl], ds[1][1][sl], ds[1][2][sl]
      x2, y2, z2 = ds[2][0][sl], ds[2][1][sl], ds[2][2][sl]
      sv = (x0 * (y1 * z2 - z1 * y2)
            + y0 * (z1 * x2 - x1 * z2)
            + z0 * (x1 * y2 - y1 * x2))
      return acc + sv

    acc = lax.fori_loop(0, CHUNKS, chunk_body,
                        jnp.zeros((L,), jnp.float32), unroll=2)
    return loss_acc + jnp.abs(jnp.sum(acc))

  loss = lax.fori_loop(0, ROWS_PW, row_body, jnp.float32(0.0))

  lane = lax.iota(jnp.int32, L)
  obuf[...] = jnp.where(lane == 0, loss * _W, jnp.float32(0.0))
  pltpu.sync_copy(obuf, out_hbm.at[wid])


@jax.jit
def kernel(final_v, ff, padded_tensor):
  fvT = final_v.T
  ffT = ff.T
  vx, vy, vz = fvT[0], fvT[1], fvT[2]
  f0, f1, f2 = ffT[0], ffT[1], ffT[2]
  padded_flat = padded_tensor.reshape(-1)
  mesh = plsc.VectorSubcoreMesh(core_axis_name="c", subcore_axis_name="s")
  i32buf = pltpu.VMEM((FPW,), jnp.int32)
  f32buf = pltpu.VMEM((FPW,), jnp.float32)
  partials = pl.kernel(
      _sc_body,
      out_type=jax.ShapeDtypeStruct((NW, L), jnp.float32),
      mesh=mesh,
      scratch_types=(
          [i32buf] * 4 + [f32buf] * 9
          + [pltpu.VMEM((L,), jnp.float32), pltpu.SemaphoreType.DMA]
      ),
      compiler_params=pltpu.CompilerParams(needs_layout_passes=False),
  )(vx, vy, vz, f0, f1, f2, padded_flat)
  return jnp.sum(partials)
